# baseline (speedup 1.0000x reference)
import jax
import jax.numpy as jnp
from jax.experimental import pallas as pl
from jax.experimental.pallas import tpu as pltpu

_NB = 64
_BW = 32
_E = 16384

_VROWS = _BW + 1
_V0 = 3 * _BW
_TROWS = _V0 + 2 * _BW + 2

_EXP_MASK = -65536


def _kernel(x_ref, bpr_ref, v_ref, o_ref, bnd_s, tab_s):
    f32 = jnp.float32
    bf16 = jnp.bfloat16

    @pl.when(pl.program_id(0) == 0)
    def _prep():
        def split3(a):
            bits = jax.lax.bitcast_convert_type(a, jnp.int32)
            hi = jax.lax.bitcast_convert_type(bits & _EXP_MASK, f32)
            r1 = a - hi
            b1 = jax.lax.bitcast_convert_type(r1, jnp.int32)
            mid = jax.lax.bitcast_convert_type(b1 & _EXP_MASK, f32)
            lo = r1 - mid
            return hi, mid, lo

        def split2(a):
            bits = jax.lax.bitcast_convert_type(a, jnp.int32)
            hi = jax.lax.bitcast_convert_type(bits & _EXP_MASK, f32)
            return hi, a - hi

        bp_t = jnp.transpose(bpr_ref[...])
        col = v_ref[...]
        vmain = jnp.transpose(col[: _NB * _BW].reshape(_NB, _BW))
        last_row = jnp.concatenate(
            [vmain[0:1, 1:], col[_NB * _BW :].reshape(1, 1)], axis=1
        )
        vt = jnp.concatenate([vmain, last_row], axis=0)

        bh, bm, bl = split3(bp_t)
        dv = vt[1:_VROWS] - vt[: _BW]
        d1, d2 = split2(dv)
        v01, v02 = split2(vt[0:1])
        table_t = jnp.concatenate([bh, bm, bl, d1, d2, v01, v02], axis=0)
        tab_s[...] = table_t.astype(bf16)

        bnd_s[...] = jnp.broadcast_to(bpr_ref[:, _BW - 1 : _BW], (_NB, _E))

    xrow = x_ref[0]
    h1 = (xrow > bnd_s[...]).astype(bf16)
    h1p = jnp.concatenate([jnp.ones((1, _E), bf16), h1[: _NB - 1]], axis=0)
    onehot = h1p - h1

    g = jnp.dot(tab_s[...], onehot, preferred_element_type=f32)
    bp_row = (g[0:_BW] + g[_BW : 2 * _BW]) + g[2 * _BW : 3 * _BW]
    dv = g[_V0 : _V0 + _BW] + g[_V0 + _BW : _V0 + 2 * _BW]
    v0 = g[_V0 + 2 * _BW : _V0 + 2 * _BW + 1] + g[_V0 + 2 * _BW + 1 : _TROWS]

    cmp = (xrow > bp_row).astype(f32)
    sel = v0 + jnp.sum(cmp * dv, axis=0, keepdims=True)

    bp_last = bpr_ref[_NB - 1, _BW - 1]
    v_last = v_ref[_NB * _BW, 0]
    out = sel + (xrow > bp_last).astype(f32) * v_last
    o_ref[...] = out.reshape(1, 1, _E)


def kernel(x, breakpoints, values):
    B = x.shape[0]
    n = breakpoints.shape[0]
    steps = B // _E

    bp_r = breakpoints.reshape(_NB, _BW)
    x3 = x.reshape(steps, 1, _E)

    out = pl.pallas_call(
        _kernel,
        out_shape=jax.ShapeDtypeStruct((steps, 1, _E), jnp.float32),
        grid=(steps,),
        in_specs=[
            pl.BlockSpec((1, 1, _E), lambda i: (i, 0, 0)),
            pl.BlockSpec((_NB, _BW), lambda i: (0, 0)),
            pl.BlockSpec((_NB * _BW + 1, 1), lambda i: (0, 0)),
        ],
        out_specs=pl.BlockSpec((1, 1, _E), lambda i: (i, 0, 0)),
        scratch_shapes=[
            pltpu.VMEM((_NB, _E), jnp.float32),
            pltpu.VMEM((_TROWS, _NB), jnp.bfloat16),
        ],
        compiler_params=pltpu.CompilerParams(
            dimension_semantics=("arbitrary",),
        ),
        name="stepnet_lookup",
    )(x3, bp_r, values)
    return out.reshape(B, 1)


# speedup vs baseline: 1.1242x; 1.1242x over previous
import jax
import jax.numpy as jnp
from jax.experimental import pallas as pl
from jax.experimental.pallas import tpu as pltpu

_NB = 64
_BW = 32
_E = 16384

_VROWS = _BW + 1
_V0 = 3 * _BW
_TROWS = _V0 + 2 * _BW + 2

_EXP_MASK = -65536


def _kernel(x_ref, bpt_ref, vt_ref, o_ref, bnd_s, tab_s):
    f32 = jnp.float32
    bf16 = jnp.bfloat16

    @pl.when(pl.program_id(0) == 0)
    def _prep():
        def split3(a):
            bits = jax.lax.bitcast_convert_type(a, jnp.int32)
            hi = jax.lax.bitcast_convert_type(bits & _EXP_MASK, f32)
            r1 = a - hi
            b1 = jax.lax.bitcast_convert_type(r1, jnp.int32)
            mid = jax.lax.bitcast_convert_type(b1 & _EXP_MASK, f32)
            lo = r1 - mid
            return hi, mid, lo

        def split2(a):
            bits = jax.lax.bitcast_convert_type(a, jnp.int32)
            hi = jax.lax.bitcast_convert_type(bits & _EXP_MASK, f32)
            return hi, a - hi

        bh, bm, bl = split3(bpt_ref[...])
        vt = vt_ref[...]
        dv = vt[1:_VROWS] - vt[: _BW]
        d1, d2 = split2(dv)
        v01, v02 = split2(vt[0:1])
        table_t = jnp.concatenate([bh, bm, bl, d1, d2, v01, v02], axis=0)
        tab_s[...] = table_t.astype(bf16)

        bnd_col = jnp.transpose(bpt_ref[_BW - 1 : _BW, :])
        bnd_s[...] = jnp.broadcast_to(bnd_col, (_NB, _E))

    xrow = x_ref[0]
    h1 = (xrow > bnd_s[...]).astype(bf16)
    h1p = jnp.concatenate([jnp.ones((1, _E), bf16), h1[: _NB - 1]], axis=0)
    onehot = h1p - h1

    g = jnp.dot(tab_s[...], onehot, preferred_element_type=f32)
    bp_row = (g[0:_BW] + g[_BW : 2 * _BW]) + g[2 * _BW : 3 * _BW]
    dv = g[_V0 : _V0 + _BW] + g[_V0 + _BW : _V0 + 2 * _BW]
    v0 = g[_V0 + 2 * _BW : _V0 + 2 * _BW + 1] + g[_V0 + 2 * _BW + 1 : _TROWS]

    cmp = (xrow > bp_row).astype(f32)
    sel = v0 + jnp.sum(cmp * dv, axis=0, keepdims=True)

    bp_last = bpt_ref[_BW - 1, _NB - 1]
    v_last = vt_ref[_BW, _NB - 1]
    out = sel + (xrow > bp_last).astype(f32) * v_last
    o_ref[...] = out.reshape(1, 1, _E)


def kernel(x, breakpoints, values):
    B = x.shape[0]
    n = breakpoints.shape[0]
    steps = B // _E

    bp_r = breakpoints.reshape(_NB, _BW)
    bp_t = bp_r.T
    v_main = values[:n, 0].reshape(_NB, _BW)
    v_ext = values[1 : n + 1, 0].reshape(_NB, _BW)
    v_t = jnp.concatenate([v_main, v_ext[:, _BW - 1 :]], axis=1).T

    x3 = x.reshape(steps, 1, _E)

    out = pl.pallas_call(
        _kernel,
        out_shape=jax.ShapeDtypeStruct((steps, 1, _E), jnp.float32),
        grid=(steps,),
        in_specs=[
            pl.BlockSpec((1, 1, _E), lambda i: (i, 0, 0)),
            pl.BlockSpec((_BW, _NB), lambda i: (0, 0)),
            pl.BlockSpec((_VROWS, _NB), lambda i: (0, 0)),
        ],
        out_specs=pl.BlockSpec((1, 1, _E), lambda i: (i, 0, 0)),
        scratch_shapes=[
            pltpu.VMEM((_NB, _E), jnp.float32),
            pltpu.VMEM((_TROWS, _NB), jnp.bfloat16),
        ],
        compiler_params=pltpu.CompilerParams(
            dimension_semantics=("arbitrary",),
        ),
        name="stepnet_lookup",
    )(x3, bp_t, v_t)
    return out.reshape(B, 1)
